# baseline (speedup 1.0000x reference)
import functools

import jax
import jax.numpy as jnp
from jax import lax
from jax.experimental import pallas as pl
from jax.experimental.pallas import tpu as pltpu
from jax.experimental.pallas import tpu_sc as plsc

_CYCLE_LEN = 288
_NUM_NODES = 10000
_NB = 1024
_NT = 12
_NW = 32
_NG = _NUM_NODES // 8
_NCHUNK_FULL = 78
_BBLK = 256
_NBLK = _NB // _BBLK
_NKB = _NT * _NBLK


def _sc_body(idx_hbm, table_hbm, out_hbm, idx_v, tbl_v, stage, ws0, ws1):
    wsems = (ws0, ws1)
    wid = lax.axis_index("s") * 2 + lax.axis_index("c")
    pltpu.sync_copy(idx_hbm, idx_v)

    def fill_block(k, buf, width):
        t = k // _NBLK
        h = k - t * _NBLK

        def b_iter(i, carry):
            bgl = i // 8
            bl0 = (i - bgl * 8) * 16
            idx16 = idx_v[t, pl.ds(h * _BBLK + i * 16, 16)]
            for w in range(width):
                col = jnp.full((16,), w, jnp.int32)
                v = plsc.load_gather(tbl_v, [idx16, col])
                stage[buf, w // 8, bgl, w - (w // 8) * 8, pl.ds(bl0, 16)] = v
            return carry

        lax.fori_loop(0, _BBLK // 16, b_iter, 0)

    def write_block(k, buf, width, ngbase, do_wait):
        t = k // _NBLK
        h = k - t * _NBLK
        ngw = width // 8
        src = stage.at[buf] if ngw == 16 else stage.at[buf, pl.ds(0, ngw)]
        c = pltpu.make_async_copy(
            src,
            out_hbm.at[t, pl.ds(ngbase, ngw), pl.ds(2 * h, 2)],
            wsems[buf])
        c.wait() if do_wait else c.start()

    def do_chunk(colbase, width):
        pltpu.sync_copy(table_hbm.at[:, pl.ds(colbase, width)],
                        tbl_v.at[:, pl.ds(0, width)])
        ngbase = colbase // 8
        fill_block(0, 0, width)
        write_block(0, 0, width, ngbase, False)
        fill_block(1, 1, width)
        write_block(1, 1, width, ngbase, False)

        def k2_body(k2, carry):
            k = 2 * k2 + 2
            write_block(k - 2, 0, width, ngbase, True)
            fill_block(k, 0, width)
            write_block(k, 0, width, ngbase, False)
            write_block(k - 1, 1, width, ngbase, True)
            fill_block(k + 1, 1, width)
            write_block(k + 1, 1, width, ngbase, False)
            return carry

        lax.fori_loop(0, (_NKB - 2) // 2, k2_body, 0)
        write_block(_NKB - 2, 0, width, ngbase, True)
        write_block(_NKB - 1, 1, width, ngbase, True)

    for ci in range(3):
        c = wid + ci * _NW

        @pl.when(c < _NCHUNK_FULL)
        def _():
            do_chunk(c * 128, 128)

    @pl.when(wid == 14)
    def _():
        do_chunk(_NCHUNK_FULL * 128, _NUM_NODES - _NCHUNK_FULL * 128)


def kernel(index, data):
    idx_t = jnp.transpose(index.astype(jnp.int32))
    mesh = plsc.VectorSubcoreMesh(core_axis_name="c", subcore_axis_name="s")
    run = functools.partial(
        pl.kernel,
        mesh=mesh,
        out_type=jax.ShapeDtypeStruct((_NT, _NG, 8, 8, 128), jnp.float32),
        scratch_types=[
            pltpu.VMEM((_NT, _NB), jnp.int32),
            pltpu.VMEM((_CYCLE_LEN, 128), jnp.float32),
            pltpu.VMEM((2, 16, 2, 8, 128), jnp.float32),
            pltpu.SemaphoreType.DMA,
            pltpu.SemaphoreType.DMA,
        ],
        compiler_params=pltpu.CompilerParams(
            use_tc_tiling_on_sc=False, needs_layout_passes=False),
    )(_sc_body)
    out5 = run(idx_t, data)
    return out5.transpose(2, 4, 0, 1, 3).reshape(_NB, _NT, _NUM_NODES)


# speedup vs baseline: 1.5065x; 1.5065x over previous
import functools

import jax
import jax.numpy as jnp
from jax import lax
from jax.experimental import pallas as pl
from jax.experimental.pallas import tpu as pltpu
from jax.experimental.pallas import tpu_sc as plsc

_CYCLE_LEN = 288
_NUM_NODES = 10000
_NB = 1024
_NT = 12
_NW = 32
_NG = _NUM_NODES // 8
_Q = _NG * 8 * 8 * 128
_NCHUNK_FULL = 78
_BBLK = 256
_NBLK = _NB // _BBLK
_NKB = _NT * _NBLK


def _sc_body(idx_hbm, table_hbm, out_hbm, idx_v, tbl_v, stage, ws0, ws1):
    wsems = (ws0, ws1)
    wid = lax.axis_index("s") * 2 + lax.axis_index("c")
    pltpu.sync_copy(idx_hbm, idx_v)

    lane = lax.iota(jnp.int32, 16)
    wconst = [
        (wg * 2 + lane // 8) * 2048 + (lane % 8) * 128 for wg in range(8)
    ]

    def fill_block(k, buf, ngroups):
        t = k // _NBLK
        h = k - t * _NBLK

        @plsc.parallel_loop(0, _BBLK // 16, unroll=2)
        def _(ii):
            idx16 = idx_v[t, pl.ds(h * _BBLK + ii * 16, 16)]
            for l in range(16):
                idx_s = idx16[l]
                bb = ii * 16 + l
                bg = bb // 128
                d = bg * 1024 + (bb - bg * 128)
                dsp = jnp.full((16,), d, jnp.int32)
                for wg in range(ngroups):
                    v = tbl_v[idx_s, pl.ds(wg * 16, 16)]
                    plsc.store_scatter(stage.at[buf], [wconst[wg] + dsp], v)

    def write_block(k, buf, ngw, ngbase, do_wait):
        t = k // _NBLK
        h = k - t * _NBLK
        for ng_l in range(ngw):
            c = pltpu.make_async_copy(
                stage.at[buf, pl.ds(ng_l * 2048, 2048)],
                out_hbm.at[t, pl.ds((ngbase + ng_l) * 8192 + h * 2048, 2048)],
                wsems[buf])
            c.wait() if do_wait else c.start()

    def do_chunk(colbase, width):
        pltpu.sync_copy(table_hbm.at[:, pl.ds(colbase, width)],
                        tbl_v.at[:, pl.ds(0, width)])
        ngbase = colbase // 8
        ngroups = width // 16
        ngw = width // 8

        def k2_body(k2, carry):
            k = 2 * k2

            @pl.when(k2 > 0)
            def _():
                write_block(k - 2, 0, ngw, ngbase, True)

            fill_block(k, 0, ngroups)
            write_block(k, 0, ngw, ngbase, False)

            @pl.when(k2 > 0)
            def _():
                write_block(k - 1, 1, ngw, ngbase, True)

            fill_block(k + 1, 1, ngroups)
            write_block(k + 1, 1, ngw, ngbase, False)
            return carry

        lax.fori_loop(0, _NKB // 2, k2_body, 0)
        write_block(_NKB - 2, 0, ngw, ngbase, True)
        write_block(_NKB - 1, 1, ngw, ngbase, True)

    def ci_body(ci, carry):
        c = wid + ci * _NW

        @pl.when(c < _NCHUNK_FULL)
        def _():
            do_chunk(c * 128, 128)

        return carry

    lax.fori_loop(0, 3, ci_body, 0)

    @pl.when(wid == 14)
    def _():
        do_chunk(_NCHUNK_FULL * 128, _NUM_NODES - _NCHUNK_FULL * 128)


def kernel(index, data):
    idx_t = jnp.transpose(index.astype(jnp.int32))
    mesh = plsc.VectorSubcoreMesh(core_axis_name="c", subcore_axis_name="s")
    run = functools.partial(
        pl.kernel,
        mesh=mesh,
        out_type=jax.ShapeDtypeStruct((_NT, _Q), jnp.float32),
        scratch_types=[
            pltpu.VMEM((_NT, _NB), jnp.int32),
            pltpu.VMEM((_CYCLE_LEN, 128), jnp.float32),
            pltpu.VMEM((2, 16 * 2048), jnp.float32),
            pltpu.SemaphoreType.DMA,
            pltpu.SemaphoreType.DMA,
        ],
        compiler_params=pltpu.CompilerParams(
            use_tc_tiling_on_sc=False, needs_layout_passes=False),
    )(_sc_body)
    out2 = run(idx_t, data)
    out5 = out2.reshape(_NT, _NG, 8, 8, 128)
    return out5.transpose(2, 4, 0, 1, 3).reshape(_NB, _NT, _NUM_NODES)


# speedup vs baseline: 3.8486x; 2.5547x over previous
import functools

import jax
import jax.numpy as jnp
from jax import lax
from jax.experimental import pallas as pl
from jax.experimental.pallas import tpu as pltpu
from jax.experimental.pallas import tpu_sc as plsc

_CYCLE_LEN = 288
_NUM_NODES = 10000
_WPAD = 10112
_WBULK = 9984
_WTAIL = _NUM_NODES - _WBULK
_NB = 1024
_NT = 12
_NW = 32
_BPW = _NB // _NW


def _sc_gather_body(idx_hbm, table_hbm, out_hbm, idx_v, g8, g4, t8, t4,
                    gs8, gs4, ws8, ws4, ts8, ts4):
    wid = lax.axis_index("s") * 2 + lax.axis_index("c")
    pltpu.sync_copy(idx_hbm.at[wid], idx_v)

    def gather8(j):
        pltpu.async_copy(table_hbm.at[idx_v.at[j, pl.ds(0, 8)]], g8, gs8)

    def gather4(j):
        pltpu.async_copy(table_hbm.at[idx_v.at[j, pl.ds(8, 4)]], g4, gs4)

    def wait_gather8(j):
        pltpu.make_async_copy(
            table_hbm.at[idx_v.at[j, pl.ds(0, 8)]], g8, gs8).wait()

    def wait_gather4(j):
        pltpu.make_async_copy(
            table_hbm.at[idx_v.at[j, pl.ds(8, 4)]], g4, gs4).wait()

    def bulk8(j, do_wait):
        bb = wid * _BPW + j
        c = pltpu.make_async_copy(
            g8.at[:, pl.ds(0, _WBULK)],
            out_hbm.at[bb, pl.ds(0, 8), pl.ds(0, _WBULK)], ws8)
        c.wait() if do_wait else c.start()

    def bulk4(j, do_wait):
        bb = wid * _BPW + j
        c = pltpu.make_async_copy(
            g4.at[:, pl.ds(0, _WBULK)],
            out_hbm.at[bb, pl.ds(8, 4), pl.ds(0, _WBULK)], ws4)
        c.wait() if do_wait else c.start()

    def tail8(j, do_wait):
        bb = wid * _BPW + j
        c = pltpu.make_async_copy(
            t8, out_hbm.at[bb, pl.ds(0, 8), pl.ds(_WBULK, _WTAIL)], ts8)
        c.wait() if do_wait else c.start()

    def tail4(j, do_wait):
        bb = wid * _BPW + j
        c = pltpu.make_async_copy(
            t4, out_hbm.at[bb, pl.ds(8, 4), pl.ds(_WBULK, _WTAIL)], ts4)
        c.wait() if do_wait else c.start()

    def step(j, first, last):
        wait_gather8(j)
        if not first:
            tail8(j - 1, True)
        for r in range(8):
            t8[r, :] = g8[r, pl.ds(_WBULK, _WTAIL)]
        bulk8(j, False)
        tail8(j, False)
        wait_gather4(j)
        if not first:
            tail4(j - 1, True)
        for r in range(4):
            t4[r, :] = g4[r, pl.ds(_WBULK, _WTAIL)]
        bulk4(j, False)
        tail4(j, False)
        bulk8(j, True)
        if not last:
            gather8(j + 1)
        bulk4(j, True)
        if not last:
            gather4(j + 1)

    gather8(0)
    gather4(0)
    step(0, True, False)
    lax.fori_loop(1, _BPW - 1, lambda j, c: (step(j, False, False), c)[1], 0)
    step(_BPW - 1, False, True)
    tail8(_BPW - 1, True)
    tail4(_BPW - 1, True)


def kernel(index, data):
    idx = index.astype(jnp.int32)
    table = jnp.pad(data, ((0, 0), (0, _WPAD - _NUM_NODES)))
    mesh = plsc.VectorSubcoreMesh(core_axis_name="c", subcore_axis_name="s")
    run = functools.partial(
        pl.kernel,
        mesh=mesh,
        out_type=jax.ShapeDtypeStruct((_NB, _NT, _NUM_NODES), jnp.float32),
        scratch_types=[
            pltpu.VMEM((_BPW, _NT), jnp.int32),
            pltpu.VMEM((8, _WPAD), jnp.float32),
            pltpu.VMEM((4, _WPAD), jnp.float32),
            pltpu.VMEM((8, _WTAIL), jnp.float32),
            pltpu.VMEM((4, _WTAIL), jnp.float32),
            pltpu.SemaphoreType.DMA,
            pltpu.SemaphoreType.DMA,
            pltpu.SemaphoreType.DMA,
            pltpu.SemaphoreType.DMA,
            pltpu.SemaphoreType.DMA,
            pltpu.SemaphoreType.DMA,
        ],
        compiler_params=pltpu.CompilerParams(use_tc_tiling_on_sc=True),
    )(_sc_gather_body)
    return run(idx.reshape(_NW, _BPW, _NT), table)
